# baseline (speedup 1.0000x reference)
import jax
import jax.numpy as jnp
from jax import lax
from jax.experimental import pallas as pl
from jax.experimental.pallas import tpu as pltpu
from jax.experimental.pallas import tpu_sc as plsc

M = 4096
D = 256
L = 16
NC = 2
NS = 16
NW = NC * NS
RPW = M // NW


def _som_update(x_hbm, emb_hbm, adj_hbm, lr_hbm, idxv_hbm, idx1_hbm, out_hbm,
                x_v, emb_v, wrow_v, lr_v, idxv_v, idx1_v, sem):
    wid = lax.axis_index("s") * NC + lax.axis_index("c")
    base = wid * RPW
    pltpu.sync_copy(lr_hbm, lr_v)
    pltpu.sync_copy(idxv_hbm, idxv_v)
    pltpu.sync_copy(idx1_hbm, idx1_v)
    pltpu.sync_copy(x_hbm, x_v)
    gat = pltpu.async_copy(adj_hbm.at[idx1_v], wrow_v, sem)
    pltpu.sync_copy(emb_hbm.at[pl.ds(base * D, RPW * D)], emb_v)
    gat.wait()
    lrv = lr_v[...]
    idxv = idxv_v[...]
    xs = [x_v[pl.ds(k * L, L)] for k in range(D // L)]

    def group_body(g, carry):
        w16 = wrow_v[0, pl.ds(base + g * L, L)]
        rows = base + g * L + lax.broadcasted_iota(jnp.int32, (L,), 0)
        is_idx = rows == idxv
        c16 = lrv * jnp.where(is_idx, jnp.float32(1.0),
                              jnp.where(w16 > jnp.float32(0.0), w16,
                                        jnp.float32(0.0)))
        for t in range(L):
            cb = jnp.full((L,), c16[t], jnp.float32)
            off = (g * L + t) * D
            for k in range(D // L):
                e = emb_v[pl.ds(off + k * L, L)]
                emb_v[pl.ds(off + k * L, L)] = e + cb * (xs[k] - e)
        return carry

    lax.fori_loop(0, RPW // L, group_body, 0)
    pltpu.sync_copy(emb_v, out_hbm.at[pl.ds(base * D, RPW * D)])


def kernel(x, embedding_to_map, embedding_to_map_adj, iter, idx, max_iter):
    lr = jnp.float32(0.1) * (jnp.float32(1.0)
                             - jnp.float32(iter) / jnp.float32(max_iter))
    idx32 = jnp.asarray(idx, jnp.int32)
    lr_arr = jnp.full((L,), lr, jnp.float32)
    idxv_arr = jnp.full((L,), idx32, jnp.int32)
    idx1_arr = jnp.full((1,), idx32, jnp.int32)
    emb_flat = embedding_to_map.reshape(M * D)
    mesh = plsc.VectorSubcoreMesh(core_axis_name="c", subcore_axis_name="s")
    som = pl.kernel(
        _som_update,
        out_type=jax.ShapeDtypeStruct((M * D,), jnp.float32),
        mesh=mesh,
        scratch_types=[
            pltpu.VMEM((D,), jnp.float32),
            pltpu.VMEM((RPW * D,), jnp.float32),
            pltpu.VMEM((1, M), jnp.float32),
            pltpu.VMEM((L,), jnp.float32),
            pltpu.VMEM((L,), jnp.int32),
            pltpu.VMEM((1,), jnp.int32),
            pltpu.SemaphoreType.DMA,
        ],
    )
    out = som(x, emb_flat, embedding_to_map_adj, lr_arr, idxv_arr, idx1_arr)
    return out.reshape(M, D)


# speedup vs baseline: 1.4035x; 1.4035x over previous
import jax
import jax.numpy as jnp
from jax import lax
from jax.experimental import pallas as pl
from jax.experimental.pallas import tpu as pltpu
from jax.experimental.pallas import tpu_sc as plsc

M = 4096
D = 256
L = 16
NC = 2
NS = 16
NW = NC * NS
RPW = M // NW
CH = 16
NCH = RPW // CH
PAIRS = NCH // 2


def _som_update(x_hbm, emb_hbm, adj_hbm, pi_hbm, pf_hbm, out_hbm,
                x_v, pi_v, pf_v, wrow_v, buf_a, buf_b,
                sem_g, sem_ai, sem_bi, sem_ao, sem_bo):
    wid = lax.axis_index("s") * NC + lax.axis_index("c")
    base = wid * RPW
    pltpu.sync_copy(pi_hbm, pi_v)
    gat = pltpu.async_copy(adj_hbm.at[pi_v.at[pl.ds(0, 1)]], wrow_v, sem_g)
    pltpu.sync_copy(pf_hbm, pf_v)
    pltpu.sync_copy(x_hbm, x_v)
    pltpu.async_copy(emb_hbm.at[pl.ds(base, CH)], buf_a, sem_ai)
    pltpu.async_copy(emb_hbm.at[pl.ds(base + CH, CH)], buf_b, sem_bi)
    idxv = pi_v[pl.ds(0, L)]
    lrv = pf_v[pl.ds(0, L)]
    gat.wait()
    xs = [x_v[pl.ds(k * L, L)] for k in range(D // L)]

    def process(buf, r0):
        w16 = wrow_v[0, pl.ds(r0, L)]
        rows = r0 + lax.broadcasted_iota(jnp.int32, (L,), 0)
        c16 = lrv * jnp.where(rows == idxv, jnp.float32(1.0),
                              jnp.where(w16 > jnp.float32(0.0), w16,
                                        jnp.float32(0.0)))
        for t in range(L):
            cb = jnp.full((L,), c16[t], jnp.float32)
            for k in range(D // L):
                e = buf[t, pl.ds(k * L, L)]
                buf[t, pl.ds(k * L, L)] = e + cb * (xs[k] - e)

    def pair(it, carry):
        a0 = base + (2 * it) * CH
        b0 = a0 + CH
        pltpu.make_async_copy(emb_hbm.at[pl.ds(a0, CH)], buf_a, sem_ai).wait()
        process(buf_a, a0)
        pltpu.async_copy(buf_a, out_hbm.at[pl.ds(a0, CH)], sem_ao)
        pltpu.make_async_copy(emb_hbm.at[pl.ds(b0, CH)], buf_b, sem_bi).wait()
        process(buf_b, b0)
        pltpu.async_copy(buf_b, out_hbm.at[pl.ds(b0, CH)], sem_bo)

        @pl.when(it < PAIRS - 1)
        def _refill():
            pltpu.make_async_copy(buf_a, out_hbm.at[pl.ds(a0, CH)],
                                  sem_ao).wait()
            pltpu.async_copy(emb_hbm.at[pl.ds(a0 + 2 * CH, CH)], buf_a, sem_ai)
            pltpu.make_async_copy(buf_b, out_hbm.at[pl.ds(b0, CH)],
                                  sem_bo).wait()
            pltpu.async_copy(emb_hbm.at[pl.ds(b0 + 2 * CH, CH)], buf_b, sem_bi)

        return carry

    lax.fori_loop(0, PAIRS, pair, 0)
    last_a = base + (NCH - 2) * CH
    last_b = base + (NCH - 1) * CH
    pltpu.make_async_copy(buf_a, out_hbm.at[pl.ds(last_a, CH)], sem_ao).wait()
    pltpu.make_async_copy(buf_b, out_hbm.at[pl.ds(last_b, CH)], sem_bo).wait()


def kernel(x, embedding_to_map, embedding_to_map_adj, iter, idx, max_iter):
    lr = jnp.float32(0.1) * (jnp.float32(1.0)
                             - jnp.float32(iter) / jnp.float32(max_iter))
    idx32 = jnp.asarray(idx, jnp.int32)
    p_idx = jnp.full((L,), idx32, jnp.int32)
    p_lr = jnp.full((L,), lr, jnp.float32)
    mesh = plsc.VectorSubcoreMesh(core_axis_name="c", subcore_axis_name="s")
    som = pl.kernel(
        _som_update,
        out_type=jax.ShapeDtypeStruct((M, D), jnp.float32),
        mesh=mesh,
        scratch_types=[
            pltpu.VMEM((D,), jnp.float32),
            pltpu.VMEM((L,), jnp.int32),
            pltpu.VMEM((L,), jnp.float32),
            pltpu.VMEM((1, M), jnp.float32),
            pltpu.VMEM((CH, D), jnp.float32),
            pltpu.VMEM((CH, D), jnp.float32),
            pltpu.SemaphoreType.DMA,
            pltpu.SemaphoreType.DMA,
            pltpu.SemaphoreType.DMA,
            pltpu.SemaphoreType.DMA,
            pltpu.SemaphoreType.DMA,
        ],
    )
    return som(x, embedding_to_map, embedding_to_map_adj, p_idx, p_lr)
